# Initial kernel scaffold; baseline (speedup 1.0000x reference)
import jax
import jax.numpy as jnp
from jax.experimental import pallas as pl


def kernel(x, edge_index, batch, W1, b1, W2, b2):
    raise NotImplementedError("write your pallas kernel here")


# speedup vs baseline: 20.3675x; 20.3675x over previous
import functools

import jax
import jax.numpy as jnp
from jax import lax
from jax.experimental import pallas as pl
from jax.experimental.pallas import tpu as pltpu
from jax.experimental.pallas import tpu_sc as plsc

N = 10000
E = 320000
D = 128
H = D // 2
NP = 10240
STRIPE = NP // 16
C = 128

BN = 1024

_MESH = plsc.VectorSubcoreMesh(core_axis_name="c", subcore_axis_name="s")

_EPC = E // 2
_EPT2 = E // 32
_NC2 = _EPT2 // C
_TL2 = _EPT2 - _NC2 * C
_EPT1 = E // 16
_NC1 = _EPT1 // C
_TL1 = _EPT1 - _NC1 * C


@functools.partial(
    pl.kernel,
    out_type=jax.ShapeDtypeStruct((2, NP), jnp.float32),
    mesh=_MESH,
    scratch_types=[
        pltpu.VMEM_SHARED((NP,), jnp.float32),
        pltpu.VMEM((C,), jnp.float32),
        pltpu.VMEM((C,), jnp.int32),
        pltpu.VMEM((_TL2,), jnp.int32),
    ],
)
def _deg(dst_hbm, out_hbm, acc, ones, idx, idx_t):
    c = lax.axis_index("c")
    s = lax.axis_index("s")

    def fill(i, _):
        ones[pl.ds(i * 16, 16)] = jnp.full((16,), 1.0, jnp.float32)
        return 0

    lax.fori_loop(0, C // 16, fill, 0)
    r0 = s * STRIPE
    for j in range(0, STRIPE // C):
        pltpu.sync_copy(ones.at[pl.ds(0, C)], acc.at[pl.ds(r0 + j * C, C)])
    plsc.subcore_barrier()

    base = c * _EPC + s * _EPT2

    def body(k, _):
        pltpu.sync_copy(dst_hbm.at[pl.ds(base + k * C, C)], idx)
        pltpu.sync_copy(ones, acc.at[idx], add=True)
        return 0

    lax.fori_loop(0, _NC2, body, 0)
    pltpu.sync_copy(dst_hbm.at[pl.ds(base + _NC2 * C, _TL2)], idx_t)
    pltpu.sync_copy(ones.at[pl.ds(0, _TL2)], acc.at[idx_t], add=True)
    plsc.subcore_barrier()
    pltpu.sync_copy(acc.at[pl.ds(r0, STRIPE)], out_hbm.at[c, pl.ds(r0, STRIPE)])


@functools.partial(
    pl.kernel,
    out_type=jax.ShapeDtypeStruct((2, NP, D), jnp.float32),
    mesh=_MESH,
    scratch_types=[
        pltpu.VMEM_SHARED((NP, D), jnp.float32),
        pltpu.VMEM((C,), jnp.int32),
        pltpu.VMEM((C,), jnp.int32),
        pltpu.VMEM((C, D), jnp.float32),
        pltpu.VMEM((_TL2,), jnp.int32),
        pltpu.VMEM((_TL2,), jnp.int32),
        pltpu.VMEM((_TL2, D), jnp.float32),
    ],
)
def _prop1(src_hbm, dst_hbm, g_hbm, out_hbm, acc, srcv, dstv, rows,
           srcv_t, dstv_t, rows_t):
    c = lax.axis_index("c")
    s = lax.axis_index("s")
    r0 = s * STRIPE
    pltpu.sync_copy(g_hbm.at[pl.ds(r0, STRIPE), :], acc.at[pl.ds(r0, STRIPE), :])
    plsc.subcore_barrier()

    base = c * _EPC + s * _EPT2

    def body(k, _):
        off = base + k * C
        pltpu.sync_copy(src_hbm.at[pl.ds(off, C)], srcv)
        pltpu.sync_copy(dst_hbm.at[pl.ds(off, C)], dstv)
        pltpu.sync_copy(g_hbm.at[srcv], rows)
        pltpu.sync_copy(rows, acc.at[dstv], add=True)
        return 0

    lax.fori_loop(0, _NC2, body, 0)
    off = base + _NC2 * C
    pltpu.sync_copy(src_hbm.at[pl.ds(off, _TL2)], srcv_t)
    pltpu.sync_copy(dst_hbm.at[pl.ds(off, _TL2)], dstv_t)
    pltpu.sync_copy(g_hbm.at[srcv_t], rows_t)
    pltpu.sync_copy(rows_t, acc.at[dstv_t], add=True)
    plsc.subcore_barrier()
    pltpu.sync_copy(acc.at[pl.ds(r0, STRIPE), :], out_hbm.at[c, pl.ds(r0, STRIPE), :])


@functools.partial(
    pl.kernel,
    out_type=jax.ShapeDtypeStruct((2, NP), jnp.float32),
    mesh=_MESH,
    scratch_types=[
        pltpu.VMEM_SHARED((NP,), jnp.float32),
        pltpu.VMEM_SHARED((NP,), jnp.float32),
        pltpu.VMEM((C,), jnp.int32),
        pltpu.VMEM((C,), jnp.int32),
        pltpu.VMEM((C,), jnp.float32),
        pltpu.VMEM((_TL2,), jnp.int32),
        pltpu.VMEM((_TL2,), jnp.int32),
        pltpu.VMEM((_TL2,), jnp.float32),
    ],
)
def _prop2(src_hbm, dst_hbm, z_hbm, out_hbm, zsp, acc, srcv, dstv, vals,
           srcv_t, dstv_t, vals_t):
    c = lax.axis_index("c")
    s = lax.axis_index("s")
    r0 = s * STRIPE
    pltpu.sync_copy(z_hbm.at[pl.ds(r0, STRIPE)], zsp.at[pl.ds(r0, STRIPE)])
    pltpu.sync_copy(z_hbm.at[pl.ds(r0, STRIPE)], acc.at[pl.ds(r0, STRIPE)])
    plsc.subcore_barrier()

    base = c * _EPC + s * _EPT2

    def body(k, _):
        off = base + k * C
        pltpu.sync_copy(src_hbm.at[pl.ds(off, C)], srcv)
        pltpu.sync_copy(dst_hbm.at[pl.ds(off, C)], dstv)
        pltpu.sync_copy(zsp.at[srcv], vals)
        pltpu.sync_copy(vals, acc.at[dstv], add=True)
        return 0

    lax.fori_loop(0, _NC2, body, 0)
    off = base + _NC2 * C
    pltpu.sync_copy(src_hbm.at[pl.ds(off, _TL2)], srcv_t)
    pltpu.sync_copy(dst_hbm.at[pl.ds(off, _TL2)], dstv_t)
    pltpu.sync_copy(zsp.at[srcv_t], vals_t)
    pltpu.sync_copy(vals_t, acc.at[dstv_t], add=True)
    plsc.subcore_barrier()
    pltpu.sync_copy(acc.at[pl.ds(r0, STRIPE)], out_hbm.at[c, pl.ds(r0, STRIPE)])


def _pre_body(x_ref, w1_ref, degp_ref, g2_ref, dinv_ref, maskf_ref):
    xb = x_ref[...]
    deg = degp_ref[0, :] + degp_ref[1, :] - 1.0
    dinv = lax.rsqrt(deg)
    h = jnp.dot(xb, w1_ref[...], preferred_element_type=jnp.float32)
    g2_ref[...] = dinv[:, None] * h
    dinv_ref[...] = dinv
    maskf_ref[...] = (xb[:, 0] == 0.0).astype(jnp.float32)


_pre = pl.pallas_call(
    _pre_body,
    grid=(NP // BN,),
    in_specs=[
        pl.BlockSpec((BN, D), lambda i: (i, 0)),
        pl.BlockSpec((D, D), lambda i: (0, 0)),
        pl.BlockSpec((2, BN), lambda i: (0, i)),
    ],
    out_specs=[
        pl.BlockSpec((BN, D), lambda i: (i, 0)),
        pl.BlockSpec((BN,), lambda i: (i,)),
        pl.BlockSpec((BN,), lambda i: (i,)),
    ],
    out_shape=[
        jax.ShapeDtypeStruct((NP, D), jnp.float32),
        jax.ShapeDtypeStruct((NP,), jnp.float32),
        jax.ShapeDtypeStruct((NP,), jnp.float32),
    ],
)


def _mid_body(acc2_ref, g_ref, dinv_ref, b1_ref, w2_ref, zt_ref):
    dinv = dinv_ref[...]
    t = acc2_ref[0] + acc2_ref[1] - g_ref[...]
    t = dinv[:, None] * t + b1_ref[...][None, :]
    t = jnp.where(t >= 0, t, 0.01 * t)
    z = jnp.dot(t, w2_ref[...], preferred_element_type=jnp.float32)
    zt_ref[...] = dinv * z


_mid = pl.pallas_call(
    _mid_body,
    grid=(NP // BN,),
    in_specs=[
        pl.BlockSpec((2, BN, D), lambda i: (0, i, 0)),
        pl.BlockSpec((BN, D), lambda i: (i, 0)),
        pl.BlockSpec((BN,), lambda i: (i,)),
        pl.BlockSpec((D,), lambda i: (0,)),
        pl.BlockSpec((D,), lambda i: (0,)),
    ],
    out_specs=pl.BlockSpec((BN,), lambda i: (i,)),
    out_shape=jax.ShapeDtypeStruct((NP,), jnp.float32),
)


def _post_body(ez_ref, zt_ref, dinv_ref, maskf_ref, b2_ref, out_ref):
    t = ez_ref[0, :] + ez_ref[1, :] - zt_ref[...]
    sc = dinv_ref[...] * t + b2_ref[...]
    valid = lax.broadcasted_iota(jnp.int32, (NP,), 0) < N
    keep = jnp.logical_and(valid, maskf_ref[...] == 0.0)
    sc = jnp.where(keep, sc, -jnp.inf)
    m = jnp.max(sc)
    e = jnp.exp(sc - m)
    out_ref[...] = (e / jnp.sum(e))[:N]


_post = pl.pallas_call(
    _post_body,
    grid=(1,),
    in_specs=[
        pl.BlockSpec((2, NP), lambda i: (0, 0)),
        pl.BlockSpec((NP,), lambda i: (0,)),
        pl.BlockSpec((NP,), lambda i: (0,)),
        pl.BlockSpec((NP,), lambda i: (0,)),
        pl.BlockSpec((1,), lambda i: (0,)),
    ],
    out_specs=pl.BlockSpec((N,), lambda i: (0,)),
    out_shape=jax.ShapeDtypeStruct((N,), jnp.float32),
)


def kernel(x, edge_index, batch, W1, b1, W2, b2):
    src = edge_index[0]
    dst = edge_index[1]
    degp = _deg(dst)
    g, dinv, maskf = _pre(x, W1, degp)
    acc2 = _prop1(src, dst, g)
    zt = _mid(acc2, g, dinv, b1, W2.reshape(D))
    ez = _prop2(src, dst, zt)
    return _post(ez, zt, dinv, maskf, b2)


# speedup vs baseline: 44.8718x; 2.2031x over previous
import functools

import jax
import jax.numpy as jnp
from jax import lax
from jax.experimental import pallas as pl
from jax.experimental.pallas import tpu as pltpu
from jax.experimental.pallas import tpu_sc as plsc

N = 10000
E = 320000
D = 128
H = D // 2
NP = 10240
STRIPE = NP // 16
C = 128

BN = 1024

_MESH = plsc.VectorSubcoreMesh(core_axis_name="c", subcore_axis_name="s")

_ROWS = E // C
_ROWS_P = 2560
_RPW = _ROWS_P // 32
_PAIRS = _RPW // 2


def _prefetch_idx(buf_hbm, buf_v, start, sem):
    pltpu.async_copy(buf_hbm.at[pl.ds(start, _RPW), :], buf_v, sem)


def _wait_idx(buf_hbm, buf_v, start, sem):
    pltpu.make_async_copy(buf_hbm.at[pl.ds(start, _RPW), :], buf_v,
                          sem).wait()


@functools.partial(
    pl.kernel,
    out_type=jax.ShapeDtypeStruct((2, NP), jnp.float32),
    mesh=_MESH,
    scratch_types=[
        pltpu.VMEM_SHARED((NP,), jnp.float32),
        pltpu.VMEM((C,), jnp.float32),
        pltpu.VMEM((_RPW, C), jnp.int32),
        pltpu.SemaphoreType.DMA,
        pltpu.SemaphoreType.DMA,
        pltpu.SemaphoreType.DMA,
    ],
)
def _deg(dst_hbm, out_hbm, acc, ones, dstbuf, sidx, ss0, ss1):
    c = lax.axis_index("c")
    s = lax.axis_index("s")
    w = c * 16 + s
    start = _RPW * w
    _prefetch_idx(dst_hbm, dstbuf, start, sidx)

    def fill(i, _):
        ones[pl.ds(i * 16, 16)] = jnp.full((16,), 1.0, jnp.float32)
        return 0

    lax.fori_loop(0, C // 16, fill, 0)
    r0 = s * STRIPE
    for j in range(0, STRIPE // C):
        pltpu.sync_copy(ones.at[pl.ds(0, C)], acc.at[pl.ds(r0 + j * C, C)])
    plsc.subcore_barrier()
    _wait_idx(dst_hbm, dstbuf, start, sidx)

    sss = (ss0, ss1)

    def put(k, slot, first):
        if not first:
            pltpu.make_async_copy(ones, acc.at[dstbuf.at[k - 2]],
                                  sss[slot]).wait()
        pltpu.async_copy(ones, acc.at[dstbuf.at[k]], sss[slot], add=True)

    put(0, 0, True)
    put(1, 1, True)

    def pair(p, _):
        put(p * 2, 0, False)
        put(p * 2 + 1, 1, False)
        return 0

    lax.fori_loop(1, _PAIRS, pair, 0)

    pltpu.make_async_copy(ones, acc.at[dstbuf.at[0]], ss0).wait()
    pltpu.make_async_copy(ones, acc.at[dstbuf.at[1]], ss1).wait()
    plsc.subcore_barrier()
    pltpu.sync_copy(acc.at[pl.ds(r0, STRIPE)], out_hbm.at[c, pl.ds(r0, STRIPE)])


_SGN = _RPW // 8


@functools.partial(
    pl.kernel,
    out_type=jax.ShapeDtypeStruct((2, NP, D), jnp.float32),
    mesh=_MESH,
    scratch_types=[
        pltpu.VMEM_SHARED((NP, D), jnp.float32),
        pltpu.VMEM((8, C), jnp.int32),
        pltpu.VMEM((8, C), jnp.int32),
        pltpu.VMEM((8, C), jnp.int32),
        pltpu.VMEM((8, C), jnp.int32),
        pltpu.VMEM((8, C), jnp.int32),
        pltpu.VMEM((8, C), jnp.int32),
        pltpu.VMEM((C, D), jnp.float32),
        pltpu.VMEM((C, D), jnp.float32),
        pltpu.SemaphoreType.DMA,
        pltpu.SemaphoreType.DMA,
        pltpu.SemaphoreType.DMA,
        pltpu.SemaphoreType.DMA,
        pltpu.SemaphoreType.DMA,
        pltpu.SemaphoreType.DMA,
        pltpu.SemaphoreType.DMA,
    ],
)
def _prop1(src_hbm, dst_hbm, g_hbm, out_hbm, acc, sb0, sb1, sb2,
           db0, db1, db2, rows0, rows1, si0, si1, si2, sg0, sg1, ss0, ss1):
    c = lax.axis_index("c")
    s = lax.axis_index("s")
    w = c * 16 + s
    r0 = s * STRIPE
    start = _RPW * w
    srcbs = (sb0, sb1, sb2)
    dstbs = (db0, db1, db2)
    sis = (si0, si1, si2)
    rowbufs = (rows0, rows1)
    sgs = (sg0, sg1)
    sss = (ss0, ss1)

    def issue_load(j, sl):
        pltpu.async_copy(src_hbm.at[pl.ds(start + j * 8, 8), :], srcbs[sl],
                         sis[sl])
        pltpu.async_copy(dst_hbm.at[pl.ds(start + j * 8, 8), :], dstbs[sl],
                         sis[sl])

    def wait_load(j, sl):
        pltpu.make_async_copy(src_hbm.at[pl.ds(start + j * 8, 8), :],
                              srcbs[sl], sis[sl]).wait()
        pltpu.make_async_copy(dst_hbm.at[pl.ds(start + j * 8, 8), :],
                              dstbs[sl], sis[sl]).wait()

    def gs(slot, srcrow, dstrow, do_wait):
        if do_wait:
            pltpu.make_async_copy(rowbufs[slot], acc.at[dstrow],
                                  sss[slot]).wait()
        d = pltpu.async_copy(g_hbm.at[srcrow], rowbufs[slot], sgs[slot])
        d.wait()
        pltpu.async_copy(rowbufs[slot], acc.at[dstrow], sss[slot], add=True)

    def run_sg(sl, do_wait_first):
        for b in range(8):
            gs(b % 2, srcbs[sl].at[b], dstbs[sl].at[b],
               do_wait_first or b >= 2)

    issue_load(0, 0)
    pltpu.sync_copy(g_hbm.at[pl.ds(r0, STRIPE), :], acc.at[pl.ds(r0, STRIPE), :])
    plsc.subcore_barrier()

    wait_load(0, 0)
    issue_load(1, 1)
    run_sg(0, False)
    wait_load(1, 1)
    issue_load(2, 2)
    run_sg(1, True)

    def triple(p, _):
        for t in range(3):
            j = 2 + p * 3 + t
            sl = (2 + t) % 3
            wait_load(j, sl)
            issue_load(j + 1, (sl + 1) % 3)
            run_sg(sl, True)
        return 0

    lax.fori_loop(0, 2, triple, 0)

    wait_load(8, 2)
    issue_load(9, 0)
    run_sg(2, True)
    wait_load(9, 0)
    run_sg(0, True)

    pltpu.make_async_copy(rows0, acc.at[db0.at[6]], ss0).wait()
    pltpu.make_async_copy(rows1, acc.at[db0.at[7]], ss1).wait()
    plsc.subcore_barrier()
    pltpu.sync_copy(acc.at[pl.ds(r0, STRIPE), :], out_hbm.at[c, pl.ds(r0, STRIPE), :])


@functools.partial(
    pl.kernel,
    out_type=jax.ShapeDtypeStruct((2, NP), jnp.float32),
    mesh=_MESH,
    scratch_types=[
        pltpu.VMEM_SHARED((NP,), jnp.float32),
        pltpu.VMEM_SHARED((NP,), jnp.float32),
        pltpu.VMEM((_RPW, C), jnp.int32),
        pltpu.VMEM((_RPW, C), jnp.int32),
        pltpu.VMEM((C,), jnp.float32),
        pltpu.VMEM((C,), jnp.float32),
        pltpu.SemaphoreType.DMA,
        pltpu.SemaphoreType.DMA,
        pltpu.SemaphoreType.DMA,
        pltpu.SemaphoreType.DMA,
        pltpu.SemaphoreType.DMA,
    ],
)
def _prop2(src_hbm, dst_hbm, z_hbm, out_hbm, zsp, acc, srcbuf, dstbuf,
           vals0, vals1, sidx, sg0, sg1, ss0, ss1):
    c = lax.axis_index("c")
    s = lax.axis_index("s")
    w = c * 16 + s
    r0 = s * STRIPE
    start = _RPW * w
    _prefetch_idx(src_hbm, srcbuf, start, sidx)
    _prefetch_idx(dst_hbm, dstbuf, start, sidx)
    pltpu.sync_copy(z_hbm.at[pl.ds(r0, STRIPE)], zsp.at[pl.ds(r0, STRIPE)])
    pltpu.sync_copy(z_hbm.at[pl.ds(r0, STRIPE)], acc.at[pl.ds(r0, STRIPE)])
    plsc.subcore_barrier()
    _wait_idx(src_hbm, srcbuf, start, sidx)
    _wait_idx(dst_hbm, dstbuf, start, sidx)

    valbufs = (vals0, vals1)
    sgs = (sg0, sg1)
    sss = (ss0, ss1)

    def gs(k, slot, first):
        if not first:
            pltpu.make_async_copy(valbufs[slot], acc.at[dstbuf.at[k - 2]],
                                  sss[slot]).wait()
        d = pltpu.async_copy(zsp.at[srcbuf.at[k]], valbufs[slot], sgs[slot])
        d.wait()
        pltpu.async_copy(valbufs[slot], acc.at[dstbuf.at[k]], sss[slot],
                         add=True)

    gs(0, 0, True)
    gs(1, 1, True)

    def pair(p, _):
        gs(p * 2, 0, False)
        gs(p * 2 + 1, 1, False)
        return 0

    lax.fori_loop(1, _PAIRS, pair, 0)

    pltpu.make_async_copy(vals0, acc.at[dstbuf.at[0]], ss0).wait()
    pltpu.make_async_copy(vals1, acc.at[dstbuf.at[1]], ss1).wait()
    plsc.subcore_barrier()
    pltpu.sync_copy(acc.at[pl.ds(r0, STRIPE)], out_hbm.at[c, pl.ds(r0, STRIPE)])


def _pre_body(x_ref, w1_ref, degp_ref, g2_ref, dinv_ref, maskf_ref):
    xb = x_ref[...]
    deg = degp_ref[0, :] + degp_ref[1, :] - 1.0
    dinv = lax.rsqrt(deg)
    h = jnp.dot(xb, w1_ref[...], preferred_element_type=jnp.float32)
    g2_ref[...] = dinv[:, None] * h
    dinv_ref[...] = dinv
    maskf_ref[...] = (xb[:, 0] == 0.0).astype(jnp.float32)


_pre = pl.pallas_call(
    _pre_body,
    grid=(NP // BN,),
    in_specs=[
        pl.BlockSpec((BN, D), lambda i: (i, 0)),
        pl.BlockSpec((D, D), lambda i: (0, 0)),
        pl.BlockSpec((2, BN), lambda i: (0, i)),
    ],
    out_specs=[
        pl.BlockSpec((BN, D), lambda i: (i, 0)),
        pl.BlockSpec((BN,), lambda i: (i,)),
        pl.BlockSpec((BN,), lambda i: (i,)),
    ],
    out_shape=[
        jax.ShapeDtypeStruct((NP, D), jnp.float32),
        jax.ShapeDtypeStruct((NP,), jnp.float32),
        jax.ShapeDtypeStruct((NP,), jnp.float32),
    ],
)


def _mid_body(acc2_ref, g_ref, dinv_ref, b1_ref, w2_ref, zt_ref):
    dinv = dinv_ref[...]
    t = acc2_ref[0] + acc2_ref[1] - g_ref[...]
    t = dinv[:, None] * t + b1_ref[...][None, :]
    t = jnp.where(t >= 0, t, 0.01 * t)
    z = jnp.dot(t, w2_ref[...], preferred_element_type=jnp.float32)
    zt_ref[...] = dinv * z


_mid = pl.pallas_call(
    _mid_body,
    grid=(NP // BN,),
    in_specs=[
        pl.BlockSpec((2, BN, D), lambda i: (0, i, 0)),
        pl.BlockSpec((BN, D), lambda i: (i, 0)),
        pl.BlockSpec((BN,), lambda i: (i,)),
        pl.BlockSpec((D,), lambda i: (0,)),
        pl.BlockSpec((D,), lambda i: (0,)),
    ],
    out_specs=pl.BlockSpec((BN,), lambda i: (i,)),
    out_shape=jax.ShapeDtypeStruct((NP,), jnp.float32),
)


def _post_body(ez_ref, zt_ref, dinv_ref, maskf_ref, b2_ref, out_ref):
    t = ez_ref[0, :] + ez_ref[1, :] - zt_ref[...]
    sc = dinv_ref[...] * t + b2_ref[...]
    valid = lax.broadcasted_iota(jnp.int32, (NP,), 0) < N
    keep = jnp.logical_and(valid, maskf_ref[...] == 0.0)
    sc = jnp.where(keep, sc, -jnp.inf)
    m = jnp.max(sc)
    e = jnp.exp(sc - m)
    out_ref[...] = (e / jnp.sum(e))[:N]


_post = pl.pallas_call(
    _post_body,
    grid=(1,),
    in_specs=[
        pl.BlockSpec((2, NP), lambda i: (0, 0)),
        pl.BlockSpec((NP,), lambda i: (0,)),
        pl.BlockSpec((NP,), lambda i: (0,)),
        pl.BlockSpec((NP,), lambda i: (0,)),
        pl.BlockSpec((1,), lambda i: (0,)),
    ],
    out_specs=pl.BlockSpec((N,), lambda i: (0,)),
    out_shape=jax.ShapeDtypeStruct((N,), jnp.float32),
)


def kernel(x, edge_index, batch, W1, b1, W2, b2):
    npad = (_ROWS_P - _ROWS) * C
    pad = (N + jnp.arange(npad, dtype=jnp.int32) % (NP - N)).reshape(
        _ROWS_P - _ROWS, C)
    ei2 = jnp.concatenate(
        [edge_index.reshape(2, _ROWS, C),
         jnp.broadcast_to(pad[None], (2, _ROWS_P - _ROWS, C))], axis=1)
    src2 = ei2[0]
    dst2 = ei2[1]
    degp = _deg(dst2)
    g, dinv, maskf = _pre(x, W1, degp)
    acc2 = _prop1(src2, dst2, g)
    zt = _mid(acc2, g, dinv, b1, W2.reshape(D))
    ez = _prop2(src2, dst2, zt)
    return _post(ez, zt, dinv, maskf, b2)


# speedup vs baseline: 45.1226x; 1.0056x over previous
import functools

import jax
import jax.numpy as jnp
from jax import lax
from jax.experimental import pallas as pl
from jax.experimental.pallas import tpu as pltpu
from jax.experimental.pallas import tpu_sc as plsc

N = 10000
E = 320000
D = 128
H = D // 2
NP = 10240
STRIPE = NP // 16
C = 128

BN = 1024

_MESH = plsc.VectorSubcoreMesh(core_axis_name="c", subcore_axis_name="s")

_ROWS = E // C
_ROWS_P = 2560
_RPW = _ROWS_P // 32
_PAIRS = _RPW // 2


def _prefetch_idx(buf_hbm, buf_v, start, sem):
    pltpu.async_copy(buf_hbm.at[pl.ds(start, _RPW), :], buf_v, sem)


def _wait_idx(buf_hbm, buf_v, start, sem):
    pltpu.make_async_copy(buf_hbm.at[pl.ds(start, _RPW), :], buf_v,
                          sem).wait()


@functools.partial(
    pl.kernel,
    out_type=jax.ShapeDtypeStruct((2, NP), jnp.float32),
    mesh=_MESH,
    scratch_types=[
        pltpu.VMEM_SHARED((NP,), jnp.float32),
        pltpu.VMEM((C,), jnp.float32),
        pltpu.VMEM((_RPW, C), jnp.int32),
        pltpu.SemaphoreType.DMA,
        pltpu.SemaphoreType.DMA,
        pltpu.SemaphoreType.DMA,
    ],
)
def _deg(dst_hbm, out_hbm, acc, ones, dstbuf, sidx, ss0, ss1):
    c = lax.axis_index("c")
    s = lax.axis_index("s")
    w = c * 16 + s
    start = _RPW * w
    _prefetch_idx(dst_hbm, dstbuf, start, sidx)

    def fill(i, _):
        ones[pl.ds(i * 16, 16)] = jnp.full((16,), 1.0, jnp.float32)
        return 0

    lax.fori_loop(0, C // 16, fill, 0)
    r0 = s * STRIPE
    for j in range(0, STRIPE // C):
        pltpu.sync_copy(ones.at[pl.ds(0, C)], acc.at[pl.ds(r0 + j * C, C)])
    plsc.subcore_barrier()
    _wait_idx(dst_hbm, dstbuf, start, sidx)

    sss = (ss0, ss1)

    def put(k, slot, first):
        if not first:
            pltpu.make_async_copy(ones, acc.at[dstbuf.at[k - 2]],
                                  sss[slot]).wait()
        pltpu.async_copy(ones, acc.at[dstbuf.at[k]], sss[slot], add=True)

    put(0, 0, True)
    put(1, 1, True)

    def pair(p, _):
        put(p * 2, 0, False)
        put(p * 2 + 1, 1, False)
        return 0

    lax.fori_loop(1, _PAIRS, pair, 0)

    pltpu.make_async_copy(ones, acc.at[dstbuf.at[0]], ss0).wait()
    pltpu.make_async_copy(ones, acc.at[dstbuf.at[1]], ss1).wait()
    plsc.subcore_barrier()
    pltpu.sync_copy(acc.at[pl.ds(r0, STRIPE)], out_hbm.at[c, pl.ds(r0, STRIPE)])


_SGN = _RPW // 8


@functools.partial(
    pl.kernel,
    out_type=jax.ShapeDtypeStruct((2, NP, D), jnp.float32),
    mesh=_MESH,
    scratch_types=[
        pltpu.VMEM_SHARED((NP, D), jnp.float32),
        pltpu.VMEM((8, C), jnp.int32),
        pltpu.VMEM((8, C), jnp.int32),
        pltpu.VMEM((8, C), jnp.int32),
        pltpu.VMEM((8, C), jnp.int32),
        pltpu.VMEM((8, C), jnp.int32),
        pltpu.VMEM((8, C), jnp.int32),
        pltpu.VMEM((C, D), jnp.float32),
        pltpu.VMEM((C, D), jnp.float32),
        pltpu.SemaphoreType.DMA,
        pltpu.SemaphoreType.DMA,
        pltpu.SemaphoreType.DMA,
        pltpu.SemaphoreType.DMA,
        pltpu.SemaphoreType.DMA,
        pltpu.SemaphoreType.DMA,
        pltpu.SemaphoreType.DMA,
    ],
)
def _prop1(src_hbm, dst_hbm, g_hbm, out_hbm, acc, sb0, sb1, sb2,
           db0, db1, db2, rows0, rows1, si0, si1, si2, sg0, sg1, ss0, ss1):
    c = lax.axis_index("c")
    s = lax.axis_index("s")
    w = c * 16 + s
    r0 = s * STRIPE
    start = _RPW * w
    srcbs = (sb0, sb1, sb2)
    dstbs = (db0, db1, db2)
    sis = (si0, si1, si2)
    rowbufs = (rows0, rows1)
    sgs = (sg0, sg1)
    sss = (ss0, ss1)

    def issue_load(j, sl):
        pltpu.async_copy(src_hbm.at[pl.ds(start + j * 8, 8), :], srcbs[sl],
                         sis[sl])
        pltpu.async_copy(dst_hbm.at[pl.ds(start + j * 8, 8), :], dstbs[sl],
                         sis[sl])

    def wait_load(j, sl):
        pltpu.make_async_copy(src_hbm.at[pl.ds(start + j * 8, 8), :],
                              srcbs[sl], sis[sl]).wait()
        pltpu.make_async_copy(dst_hbm.at[pl.ds(start + j * 8, 8), :],
                              dstbs[sl], sis[sl]).wait()

    def gs(slot, srcrow, dstrow, do_wait):
        if do_wait:
            pltpu.make_async_copy(rowbufs[slot], acc.at[dstrow],
                                  sss[slot]).wait()
        d = pltpu.async_copy(g_hbm.at[srcrow], rowbufs[slot], sgs[slot])
        d.wait()
        pltpu.async_copy(rowbufs[slot], acc.at[dstrow], sss[slot], add=True)

    def run_sg(sl, do_wait_first):
        for b in range(8):
            gs(b % 2, srcbs[sl].at[b], dstbs[sl].at[b],
               do_wait_first or b >= 2)

    issue_load(0, 0)
    @pl.when(c == 0)
    def _():
        pltpu.sync_copy(g_hbm.at[pl.ds(r0, STRIPE), :],
                        acc.at[pl.ds(r0, STRIPE), :])

    @pl.when(c == 1)
    def _():
        def zfill(i, _):
            rows0[i // 8, pl.ds(lax.rem(i, 8) * 16, 16)] = jnp.zeros(
                (16,), jnp.float32)
            return 0

        lax.fori_loop(0, C * 8, zfill, 0)
        for q in range(STRIPE // C):
            pltpu.sync_copy(rows0, acc.at[pl.ds(r0 + q * C, C), :])

    plsc.subcore_barrier()

    wait_load(0, 0)
    issue_load(1, 1)
    run_sg(0, False)
    wait_load(1, 1)
    issue_load(2, 2)
    run_sg(1, True)

    def triple(p, _):
        for t in range(3):
            j = 2 + p * 3 + t
            sl = (2 + t) % 3
            wait_load(j, sl)
            issue_load(j + 1, (sl + 1) % 3)
            run_sg(sl, True)
        return 0

    lax.fori_loop(0, 2, triple, 0)

    wait_load(8, 2)
    issue_load(9, 0)
    run_sg(2, True)
    wait_load(9, 0)
    run_sg(0, True)

    pltpu.make_async_copy(rows0, acc.at[db0.at[6]], ss0).wait()
    pltpu.make_async_copy(rows1, acc.at[db0.at[7]], ss1).wait()
    plsc.subcore_barrier()
    pltpu.sync_copy(acc.at[pl.ds(r0, STRIPE), :], out_hbm.at[c, pl.ds(r0, STRIPE), :])


@functools.partial(
    pl.kernel,
    out_type=jax.ShapeDtypeStruct((2, NP), jnp.float32),
    mesh=_MESH,
    scratch_types=[
        pltpu.VMEM_SHARED((NP,), jnp.float32),
        pltpu.VMEM_SHARED((NP,), jnp.float32),
        pltpu.VMEM((_RPW, C), jnp.int32),
        pltpu.VMEM((_RPW, C), jnp.int32),
        pltpu.VMEM((C,), jnp.float32),
        pltpu.VMEM((C,), jnp.float32),
        pltpu.SemaphoreType.DMA,
        pltpu.SemaphoreType.DMA,
        pltpu.SemaphoreType.DMA,
        pltpu.SemaphoreType.DMA,
        pltpu.SemaphoreType.DMA,
    ],
)
def _prop2(src_hbm, dst_hbm, z_hbm, out_hbm, zsp, acc, srcbuf, dstbuf,
           vals0, vals1, sidx, sg0, sg1, ss0, ss1):
    c = lax.axis_index("c")
    s = lax.axis_index("s")
    w = c * 16 + s
    r0 = s * STRIPE
    start = _RPW * w
    _prefetch_idx(src_hbm, srcbuf, start, sidx)
    _prefetch_idx(dst_hbm, dstbuf, start, sidx)
    pltpu.sync_copy(z_hbm.at[pl.ds(r0, STRIPE)], zsp.at[pl.ds(r0, STRIPE)])

    @pl.when(c == 0)
    def _():
        pltpu.sync_copy(z_hbm.at[pl.ds(r0, STRIPE)], acc.at[pl.ds(r0, STRIPE)])

    @pl.when(c == 1)
    def _():
        def zfill(i, _):
            vals0[pl.ds(i * 16, 16)] = jnp.zeros((16,), jnp.float32)
            return 0

        lax.fori_loop(0, C // 16, zfill, 0)
        for q in range(STRIPE // C):
            pltpu.sync_copy(vals0, acc.at[pl.ds(r0 + q * C, C)])

    plsc.subcore_barrier()
    _wait_idx(src_hbm, srcbuf, start, sidx)
    _wait_idx(dst_hbm, dstbuf, start, sidx)

    valbufs = (vals0, vals1)
    sgs = (sg0, sg1)
    sss = (ss0, ss1)

    def gs(k, slot, first):
        if not first:
            pltpu.make_async_copy(valbufs[slot], acc.at[dstbuf.at[k - 2]],
                                  sss[slot]).wait()
        d = pltpu.async_copy(zsp.at[srcbuf.at[k]], valbufs[slot], sgs[slot])
        d.wait()
        pltpu.async_copy(valbufs[slot], acc.at[dstbuf.at[k]], sss[slot],
                         add=True)

    gs(0, 0, True)
    gs(1, 1, True)

    def pair(p, _):
        gs(p * 2, 0, False)
        gs(p * 2 + 1, 1, False)
        return 0

    lax.fori_loop(1, _PAIRS, pair, 0)

    pltpu.make_async_copy(vals0, acc.at[dstbuf.at[0]], ss0).wait()
    pltpu.make_async_copy(vals1, acc.at[dstbuf.at[1]], ss1).wait()
    plsc.subcore_barrier()
    pltpu.sync_copy(acc.at[pl.ds(r0, STRIPE)], out_hbm.at[c, pl.ds(r0, STRIPE)])


def _pre_body(x_ref, w1_ref, degp_ref, g2_ref, dinv_ref, maskf_ref):
    xb = x_ref[...]
    deg = degp_ref[0, :] + degp_ref[1, :] - 1.0
    dinv = lax.rsqrt(deg)
    h = jnp.dot(xb, w1_ref[...], preferred_element_type=jnp.float32)
    g2_ref[...] = dinv[:, None] * h
    dinv_ref[...] = dinv
    maskf_ref[...] = (xb[:, 0] == 0.0).astype(jnp.float32)


_pre = pl.pallas_call(
    _pre_body,
    grid=(NP // BN,),
    in_specs=[
        pl.BlockSpec((BN, D), lambda i: (i, 0)),
        pl.BlockSpec((D, D), lambda i: (0, 0)),
        pl.BlockSpec((2, BN), lambda i: (0, i)),
    ],
    out_specs=[
        pl.BlockSpec((BN, D), lambda i: (i, 0)),
        pl.BlockSpec((BN,), lambda i: (i,)),
        pl.BlockSpec((BN,), lambda i: (i,)),
    ],
    out_shape=[
        jax.ShapeDtypeStruct((NP, D), jnp.float32),
        jax.ShapeDtypeStruct((NP,), jnp.float32),
        jax.ShapeDtypeStruct((NP,), jnp.float32),
    ],
)


def _mid_body(acc2_ref, dinv_ref, b1_ref, w2_ref, zt_ref):
    dinv = dinv_ref[...]
    t = acc2_ref[0] + acc2_ref[1]
    t = dinv[:, None] * t + b1_ref[...][None, :]
    t = jnp.where(t >= 0, t, 0.01 * t)
    z = jnp.dot(t, w2_ref[...], preferred_element_type=jnp.float32)
    zt_ref[...] = dinv * z


_mid = pl.pallas_call(
    _mid_body,
    grid=(NP // BN,),
    in_specs=[
        pl.BlockSpec((2, BN, D), lambda i: (0, i, 0)),
        pl.BlockSpec((BN,), lambda i: (i,)),
        pl.BlockSpec((D,), lambda i: (0,)),
        pl.BlockSpec((D,), lambda i: (0,)),
    ],
    out_specs=pl.BlockSpec((BN,), lambda i: (i,)),
    out_shape=jax.ShapeDtypeStruct((NP,), jnp.float32),
)


def _post_body(ez_ref, dinv_ref, maskf_ref, b2_ref, out_ref):
    t = ez_ref[0, :] + ez_ref[1, :]
    sc = dinv_ref[...] * t + b2_ref[...]
    valid = lax.broadcasted_iota(jnp.int32, (NP,), 0) < N
    keep = jnp.logical_and(valid, maskf_ref[...] == 0.0)
    sc = jnp.where(keep, sc, -jnp.inf)
    m = jnp.max(sc)
    e = jnp.exp(sc - m)
    out_ref[...] = (e / jnp.sum(e))[:N]


_post = pl.pallas_call(
    _post_body,
    grid=(1,),
    in_specs=[
        pl.BlockSpec((2, NP), lambda i: (0, 0)),
        pl.BlockSpec((NP,), lambda i: (0,)),
        pl.BlockSpec((NP,), lambda i: (0,)),
        pl.BlockSpec((1,), lambda i: (0,)),
    ],
    out_specs=pl.BlockSpec((N,), lambda i: (0,)),
    out_shape=jax.ShapeDtypeStruct((N,), jnp.float32),
)


def kernel(x, edge_index, batch, W1, b1, W2, b2):
    npad = (_ROWS_P - _ROWS) * C
    pad = (N + jnp.arange(npad, dtype=jnp.int32) % (NP - N)).reshape(
        _ROWS_P - _ROWS, C)
    ei2 = jnp.concatenate(
        [edge_index.reshape(2, _ROWS, C),
         jnp.broadcast_to(pad[None], (2, _ROWS_P - _ROWS, C))], axis=1)
    src2 = ei2[0]
    dst2 = ei2[1]
    degp = _deg(dst2)
    g, dinv, maskf = _pre(x, W1, degp)
    acc2 = _prop1(src2, dst2, g)
    zt = _mid(acc2, dinv, b1, W2.reshape(D))
    ez = _prop2(src2, dst2, zt)
    return _post(ez, dinv, maskf, b2)
